# Initial kernel scaffold; baseline (speedup 1.0000x reference)
import jax
import jax.numpy as jnp
from jax.experimental import pallas as pl


def kernel(x, edge_index, edge_attr, W_l0, b_l0, W_r0, b_r0, W_e0, att0, bias0, W_l1, b_l1, W_r1, b_r1, W_e1, att1, bias1, W_lin, b_lin):
    raise NotImplementedError("write your pallas kernel here")


# speedup vs baseline: 6.7802x; 6.7802x over previous
import functools

import jax
import jax.numpy as jnp
from jax import lax
from jax.experimental import pallas as pl
from jax.experimental.pallas import tpu as pltpu
from jax.experimental.pallas import tpu_sc as plsc

N = 10000
E = 320000
HEADS = 4
HID = 64
DF = 128
OUT = 128
NEG = 0.2

NC, NS, L = 2, 16, 16
NP = 10240
NP2 = 2 * NP
EPAD = 327680
EPT = EPAD // NS
LPT = NP // NS
EP3 = EPAD + NP

@functools.cache
def _mesh():
    return plsc.VectorSubcoreMesh(core_axis_name="c", subcore_axis_name="s",
                                  num_cores=NC, num_subcores=NS)


_BN = 512


def _proj_body(widths, relu, *refs):
    nparts = len(widths)
    parts = refs[:nparts]
    w_ref, b_ref, o_ref = refs[nparts], refs[nparts + 1], refs[nparts + 2]
    k0 = 0
    acc = None
    for i, w in enumerate(widths):
        term = jnp.dot(parts[i][...], w_ref[k0:k0 + w, :],
                       preferred_element_type=jnp.float32)
        acc = term if acc is None else acc + term
        k0 += w
    acc = acc + b_ref[...]
    if relu:
        acc = jnp.maximum(acc, 0.0)
    o_ref[...] = acc


def _proj(parts, w, b, relu=False):
    widths = tuple(p.shape[1] for p in parts)
    dout = w.shape[1]
    grid = (NP // _BN,)
    in_specs = [pl.BlockSpec((_BN, pw), lambda i: (i, 0)) for pw in widths]
    in_specs.append(pl.BlockSpec(w.shape, lambda i: (0, 0)))
    in_specs.append(pl.BlockSpec((1, dout), lambda i: (0, 0)))
    out_shape = jax.ShapeDtypeStruct((NP, dout), jnp.float32)
    out_spec = pl.BlockSpec((_BN, dout), lambda i: (i, 0))
    return pl.pallas_call(
        functools.partial(_proj_body, widths, relu),
        grid=grid,
        in_specs=in_specs,
        out_specs=out_spec,
        out_shape=out_shape,
    )(*parts, w, b.reshape(1, dout))


_CH0 = 2048


def _k0_body(eix, ea, la, sv, dv, eav, accs, accc, shared, ts, tt, sem):
    c = lax.axis_index("c")
    s = lax.axis_index("s")

    @pl.loop(0, NP // L)
    def _(i):
        z = jnp.zeros((L,), jnp.float32)
        accs[pl.ds(i * L, L)] = z
        accc[pl.ds(i * L, L)] = z

    base = s * EPT

    @pl.loop(0, EPT // _CH0)
    def _(k):
        off = base + k * _CH0
        pltpu.sync_copy(eix.at[0, pl.ds(off, _CH0)], sv)
        pltpu.sync_copy(eix.at[1, pl.ds(off, _CH0)], dv)
        pltpu.sync_copy(ea.at[pl.ds(off, _CH0)], eav)

        @pl.loop(0, _CH0 // L)
        def _(i):
            ds = pl.ds(i * L, L)
            svv = sv[ds]
            dvv = dv[ds]
            evv = eav[ds]
            m = svv != dvv
            val = jnp.where(m, evv, 0.0)
            cnt = jnp.where(m, 1.0, 0.0)
            plsc.addupdate_scatter(accs, [dvv], val)
            plsc.addupdate_scatter(accc, [dvv], cnt)

    pltpu.sync_copy(accs, shared.at[s, 0])
    pltpu.sync_copy(accc, shared.at[s, 1])
    plsc.subcore_barrier()

    r0 = s * LPT
    pltpu.sync_copy(shared.at[0, 0, pl.ds(r0, LPT)], accs.at[pl.ds(0, LPT)])
    pltpu.sync_copy(shared.at[0, 1, pl.ds(r0, LPT)], accc.at[pl.ds(0, LPT)])
    for t in range(1, NS):
        pltpu.sync_copy(shared.at[t, 0, pl.ds(r0, LPT)], ts)
        pltpu.sync_copy(shared.at[t, 1, pl.ds(r0, LPT)], tt)

        @pl.loop(0, LPT // L)
        def _(i):
            ds = pl.ds(i * L, L)
            accs[ds] = accs[ds] + ts[ds]
            accc[ds] = accc[ds] + tt[ds]

    @pl.loop(0, LPT // L)
    def _(i):
        ds = pl.ds(i * L, L)
        ts[ds] = accs[ds] / jnp.maximum(accc[ds], 1.0)

    pltpu.sync_copy(ts.at[pl.ds(0, LPT)], la.at[c, pl.ds(r0, LPT)])


def _k0(eix, ea):
    f = pl.kernel(
        _k0_body,
        out_type=jax.ShapeDtypeStruct((2, NP), jnp.float32),
        mesh=_mesh(),
        compiler_params=pltpu.CompilerParams(needs_layout_passes=False),
        scratch_types=[
            pltpu.VMEM((_CH0,), jnp.int32),
            pltpu.VMEM((_CH0,), jnp.int32),
            pltpu.VMEM((_CH0,), jnp.float32),
            pltpu.VMEM((NP,), jnp.float32),
            pltpu.VMEM((NP,), jnp.float32),
            pltpu.VMEM_SHARED((NS, 2, NP), jnp.float32),
            pltpu.VMEM((LPT,), jnp.float32),
            pltpu.VMEM((LPT,), jnp.float32),
            pltpu.SemaphoreType.DMA,
        ],
    )
    return f(eix, ea)


_CHA = 256


def _ka_edge_chunk(c, loop_phase, sv, dv, eav, gis, gid, A, B, a0st, a1st,
                   xlr, xrr, sem, we, at0, at1, cha):
    nsub = cha // 128
    for q in range(nsub):
        pltpu.async_copy(xlr.at[gis.at[q]], A.at[pl.ds(q * 128, 128)],
                         sem).wait()
        pltpu.async_copy(xrr.at[gid.at[q]], B.at[pl.ds(q * 128, 128)],
                         sem).wait()

    liota = lax.iota(jnp.int32, L)

    @pl.loop(0, cha // L)
    def _(i):
        ds = pl.ds(i * L, L)
        ev = eav[ds]
        svv = sv[ds]
        dvv = dv[ds]
        if loop_phase:
            badv = svv >= N
        else:
            badv = svv == dvv
        av0 = jnp.zeros((L,), jnp.float32)
        av1 = jnp.zeros((L,), jnp.float32)
        for lane in range(L):
            j = i * L + lane
            eaj = ev[lane]
            acc0 = jnp.zeros((L,), jnp.float32)
            acc1 = jnp.zeros((L,), jnp.float32)
            for kk in range(8):
                dsk = pl.ds(kk * L, L)
                m = A[j, dsk] + B[j, dsk] + eaj * we[kk]
                m = jnp.maximum(m, 0.0) + NEG * jnp.minimum(m, 0.0)
                if kk < 4:
                    acc0 = acc0 + m * at0[kk]
                else:
                    acc1 = acc1 + m * at1[kk - 4]
            al0 = jnp.sum(acc0)
            al1 = jnp.sum(acc1)
            av0 = av0 + jnp.where(liota == lane, al0, 0.0)
            av1 = av1 + jnp.where(liota == lane, al1, 0.0)
        a0st[ds] = jnp.where(badv, -1e30, av0)
        a1st[ds] = jnp.where(badv, -1e30, av1)


def _ka_body(xlr, xrr, eix, ea, la, we2, att2, alpha, mx,
             sv, dv, eav, gis, gid, A, B, a0st, a1st, wev, attv, mxv,
             shmx, sem):
    c = lax.axis_index("c")
    s = lax.axis_index("s")

    pltpu.sync_copy(we2.at[c], wev)
    pltpu.sync_copy(att2.at[c], attv)
    we = [wev[pl.ds(k * L, L)] for k in range(8)]
    at0 = [attv[0, pl.ds(k * L, L)] for k in range(4)]
    at1 = [attv[1, pl.ds(k * L, L)] for k in range(4)]

    neg = jnp.full((L,), -1e30, jnp.float32)

    base = s * EPT

    @pl.loop(0, EPT // _CHA, init_carry=(neg, neg))
    def real_mx(k, carry):
        off = base + k * _CHA
        pltpu.sync_copy(eix.at[0, pl.ds(off, _CHA)], sv)
        pltpu.sync_copy(eix.at[1, pl.ds(off, _CHA)], dv)
        pltpu.sync_copy(ea.at[pl.ds(off, _CHA)], eav)
        for q in range(_CHA // 128):
            @pl.loop(0, 128 // L)
            def _(i):
                ds = pl.ds(q * 128 + i * L, L)
                dsq = pl.ds(i * L, L)
                gis[q, dsq] = sv[ds] * 2 + c
                gid[q, dsq] = dv[ds] * 2 + c
        _ka_edge_chunk(c, False, sv, dv, eav, gis, gid, A, B, a0st, a1st,
                       xlr, xrr, sem, we, at0, at1, _CHA)
        pltpu.sync_copy(a0st, alpha.at[c, 0, pl.ds(off, _CHA)])
        pltpu.sync_copy(a1st, alpha.at[c, 1, pl.ds(off, _CHA)])
        m0, m1 = carry

        @pl.loop(0, _CHA // L, init_carry=(m0, m1))
        def mxf(i, cc):
            p0, p1 = cc
            ds = pl.ds(i * L, L)
            return (jnp.maximum(p0, a0st[ds]), jnp.maximum(p1, a1st[ds]))

        return mxf

    nb = s * LPT
    liota = lax.iota(jnp.int32, L)

    @pl.loop(0, LPT // 128, init_carry=real_mx)
    def loop_mx(k, carry):
        off_n = nb + k * 128
        pltpu.sync_copy(la.at[c, pl.ds(off_n, 128)], eav.at[pl.ds(0, 128)])

        @pl.loop(0, 128 // L)
        def _(i):
            nv = off_n + i * L + liota
            ncl = jnp.minimum(nv, N - 1)
            dsq = pl.ds(i * L, L)
            sv[dsq] = nv
            dv[dsq] = nv
            gis[0, dsq] = ncl * 2 + c
            gid[0, dsq] = ncl * 2 + c
        _ka_edge_chunk(c, True, sv, dv, eav, gis, gid, A, B, a0st, a1st,
                       xlr, xrr, sem, we, at0, at1, 128)
        aoff = EPAD + off_n
        pltpu.sync_copy(a0st.at[pl.ds(0, 128)], alpha.at[c, 0, pl.ds(aoff, 128)])
        pltpu.sync_copy(a1st.at[pl.ds(0, 128)], alpha.at[c, 1, pl.ds(aoff, 128)])
        m0, m1 = carry

        @pl.loop(0, 128 // L, init_carry=(m0, m1))
        def mxf(i, cc):
            p0, p1 = cc
            ds = pl.ds(i * L, L)
            return (jnp.maximum(p0, a0st[ds]), jnp.maximum(p1, a1st[ds]))

        return mxf

    m0, m1 = loop_mx
    mxv[0, pl.ds(0, L)] = m0
    mxv[1, pl.ds(0, L)] = m1
    pltpu.sync_copy(mxv, shmx.at[s])
    plsc.subcore_barrier()

    pltpu.sync_copy(shmx.at[0], mxv)
    r0 = mxv[0, pl.ds(0, L)]
    r1 = mxv[1, pl.ds(0, L)]
    for t in range(1, NS):
        pltpu.sync_copy(shmx.at[t], mxv)
        r0 = jnp.maximum(r0, mxv[0, pl.ds(0, L)])
        r1 = jnp.maximum(r1, mxv[1, pl.ds(0, L)])
    mxv[0, pl.ds(0, L)] = jnp.full((L,), jnp.max(r0), jnp.float32)
    mxv[1, pl.ds(0, L)] = jnp.full((L,), jnp.max(r1), jnp.float32)

    @pl.when(s == 0)
    def _():
        pltpu.sync_copy(mxv, mx.at[c])


def _ka(xlr, xrr, eix, ea, la, we2, att2):
    f = pl.kernel(
        _ka_body,
        out_type=[
            jax.ShapeDtypeStruct((2, 2, EP3), jnp.float32),
            jax.ShapeDtypeStruct((2, 2, L), jnp.float32),
        ],
        mesh=_mesh(),
        compiler_params=pltpu.CompilerParams(needs_layout_passes=False),
        scratch_types=[
            pltpu.VMEM((_CHA,), jnp.int32),
            pltpu.VMEM((_CHA,), jnp.int32),
            pltpu.VMEM((_CHA,), jnp.float32),
            pltpu.VMEM((_CHA // 128, 128), jnp.int32),
            pltpu.VMEM((_CHA // 128, 128), jnp.int32),
            pltpu.VMEM((_CHA, 128), jnp.float32),
            pltpu.VMEM((_CHA, 128), jnp.float32),
            pltpu.VMEM((_CHA,), jnp.float32),
            pltpu.VMEM((_CHA,), jnp.float32),
            pltpu.VMEM((128,), jnp.float32),
            pltpu.VMEM((2, 64), jnp.float32),
            pltpu.VMEM((2, L), jnp.float32),
            pltpu.VMEM_SHARED((NS, 2, L), jnp.float32),
            pltpu.SemaphoreType.DMA,
        ],
    )
    return f(xlr, xrr, eix, ea, la, we2, att2)


_CHC = 128


def _kc_body(alpha, mx, eix, xlr, bias4, h,
             sv, dvi, gis, exv, A, As, denp, dred, zbuf, mxv, bv4, iov,
             acc, densh, sem):
    c = lax.axis_index("c")
    s = lax.axis_index("s")
    liota = lax.iota(jnp.int32, L)

    pltpu.sync_copy(mx.at[c], mxv)
    pltpu.sync_copy(bias4, bv4)

    @pl.loop(0, 80 // L)
    def _(i):
        iov[0, pl.ds(i * L, L)] = i * L + liota

    for p in range(2):
        mp = mxv[p, pl.ds(0, L)][0]
        hidx = c * 2 + p
        bq = [bv4[hidx, pl.ds(kk * L, L)] for kk in range(4)]

        @pl.loop(0, 80)
        def _(r):
            for kk in range(8):
                denp[r, pl.ds(kk * L, L)] = jnp.zeros((L,), jnp.float32)

        @pl.loop(0, 64)
        def _(j):
            for kk in range(8):
                As[j, pl.ds(kk * L, L)] = jnp.zeros((L,), jnp.float32)

        @pl.loop(0, 5)
        def _(r):
            for kk in range(8):
                zbuf[r, pl.ds(kk * L, L)] = jnp.zeros((L,), jnp.float32)
        for q in range(5):
            pltpu.sync_copy(As.at[pl.ds(0, 64)],
                            acc.at[pl.ds(s * 320 + q * 64, 64)])
        pltpu.sync_copy(zbuf, densh.at[pl.ds(s * 5, 5)])
        plsc.subcore_barrier()

        def do_chunk(off, aoff, loop_phase):
            cha = _CHC
            if loop_phase:
                @pl.loop(0, cha // L)
                def _(i):
                    nv = off + i * L + liota
                    ncl = jnp.minimum(nv, N - 1)
                    sv[pl.ds(i * L, L)] = ncl
            else:
                pltpu.sync_copy(eix.at[0, pl.ds(off, cha)], sv)
            pltpu.sync_copy(alpha.at[c, p, pl.ds(aoff, cha)], exv)

            @pl.loop(0, cha // L)
            def _(i):
                ds = pl.ds(i * L, L)
                gis[ds] = sv[ds] * 2 + c
                exv[ds] = jnp.exp(exv[ds] - mp)

            if not loop_phase:
                pltpu.sync_copy(eix.at[1, pl.ds(off, cha)], sv)

            @pl.loop(0, cha // L)
            def _(i):
                ds = pl.ds(i * L, L)
                dvv = sv[ds]
                plsc.addupdate_scatter(denp, [dvv // 128, dvv % 128], exv[ds])
                dvi[0, ds] = dvv // 2

            pltpu.async_copy(xlr.at[gis], A, sem).wait()

            @pl.loop(0, cha // L)
            def _(i):
                ds = pl.ds(i * L, L)
                ev = exv[ds]
                pf = (sv[ds] % 2).astype(jnp.float32)
                for lane in range(L):
                    j = i * L + lane
                    e = ev[lane]
                    p1 = pf[lane]
                    e0 = e * (1.0 - p1)
                    e1 = e * p1
                    for kk in range(4):
                        v = A[j, pl.ds(p * 64 + kk * L, L)]
                        As[j, pl.ds(kk * L, L)] = v * e0
                        As[j, pl.ds(64 + kk * L, L)] = v * e1

            pltpu.sync_copy(As, acc.at[dvi.at[0]], add=True)

        base = s * EPT

        @pl.loop(0, EPT // _CHC)
        def _(k):
            off = base + k * _CHC
            do_chunk(off, off, False)

        nb = s * LPT

        @pl.loop(0, LPT // _CHC)
        def _(k):
            off = nb + k * _CHC
            do_chunk(off, EPAD + off, True)

        pltpu.sync_copy(denp, densh.at[iov.at[0]], add=True)
        plsc.subcore_barrier()

        pltpu.sync_copy(densh.at[pl.ds(s * 5, 5)], dred)
        for q in range(5):
            rr = s * 320 + q * 64
            pltpu.sync_copy(acc.at[pl.ds(rr, 64)], As.at[pl.ds(0, 64)])

            @pl.loop(0, 8)
            def _(g2):
                invv = 1.0 / (dred[q, pl.ds(g2 * L, L)] + 1e-16)
                for jj in range(8):
                    j = g2 * 8 + jj
                    i0 = invv[2 * jj]
                    i1 = invv[2 * jj + 1]
                    for kk in range(4):
                        d0 = pl.ds(kk * L, L)
                        d1 = pl.ds(64 + kk * L, L)
                        As[j, d0] = jnp.maximum(As[j, d0] * i0 + bq[kk], 0.0)
                        As[j, d1] = jnp.maximum(As[j, d1] * i1 + bq[kk], 0.0)

            pltpu.sync_copy(As.at[pl.ds(0, 64)], h.at[c, p, pl.ds(rr, 64)])
        plsc.subcore_barrier()


def _kc(alpha, mx, eix, xlr, bias4):
    f = pl.kernel(
        _kc_body,
        out_type=jax.ShapeDtypeStruct((2, 2, NP // 2, 128), jnp.float32),
        mesh=_mesh(),
        compiler_params=pltpu.CompilerParams(needs_layout_passes=False),
        scratch_types=[
            pltpu.VMEM((_CHC,), jnp.int32),
            pltpu.VMEM((1, 128), jnp.int32),
            pltpu.VMEM((_CHC,), jnp.int32),
            pltpu.VMEM((_CHC,), jnp.float32),
            pltpu.VMEM((_CHC, 128), jnp.float32),
            pltpu.VMEM((_CHC, 128), jnp.float32),
            pltpu.VMEM((80, 128), jnp.float32),
            pltpu.VMEM((5, 128), jnp.float32),
            pltpu.VMEM((5, 128), jnp.float32),
            pltpu.VMEM((2, L), jnp.float32),
            pltpu.VMEM((4, 64), jnp.float32),
            pltpu.VMEM((1, 80), jnp.int32),
            pltpu.VMEM_SHARED((NP // 2, 128), jnp.float32),
            pltpu.VMEM_SHARED((80, 128), jnp.float32),
            pltpu.SemaphoreType.DMA,
        ],
    )
    return f(alpha, mx, eix, xlr, bias4)


def kernel(x, edge_index, edge_attr,
           W_l0, b_l0, W_r0, b_r0, W_e0, att0, bias0,
           W_l1, b_l1, W_r1, b_r1, W_e1, att1, bias1,
           W_lin, b_lin):
    x = x.astype(jnp.float32)
    ea = edge_attr.astype(jnp.float32).reshape(E)
    ea = jnp.pad(ea, (0, EPAD - E))
    eix = jnp.pad(edge_index, ((0, 0), (0, EPAD - E)))

    xp = jnp.pad(x, ((0, NP - N), (0, 0)))

    la = _k0(eix, ea)

    we2_0 = W_e0.reshape(2, 128)
    att2_0 = att0.reshape(2, 2, HID)
    bias4_0 = bias0.reshape(4, 64)
    we2_1 = W_e1.reshape(2, 128)
    att2_1 = att1.reshape(2, 2, HID)
    bias4_1 = bias1.reshape(4, 64)

    xl0 = _proj([xp], W_l0, b_l0)
    xr0 = _proj([xp], W_r0, b_r0)
    alpha0, mx0 = _ka(xl0.reshape(NP2, 128), xr0.reshape(NP2, 128),
                      eix, ea, la, we2_0, att2_0)
    h0 = _kc(alpha0, mx0, eix, xl0.reshape(NP2, 128), bias4_0)

    parts0 = [h0[0, 0].reshape(NP, 64), h0[0, 1].reshape(NP, 64),
              h0[1, 0].reshape(NP, 64), h0[1, 1].reshape(NP, 64)]
    xl1 = _proj(parts0, W_l1, b_l1)
    xr1 = _proj(parts0, W_r1, b_r1)
    alpha1, mx1 = _ka(xl1.reshape(NP2, 128), xr1.reshape(NP2, 128),
                      eix, ea, la, we2_1, att2_1)
    r1 = _kc(alpha1, mx1, eix, xl1.reshape(NP2, 128), bias4_1)

    w_cat = jnp.concatenate([W_lin, W_lin], axis=0)
    out = _proj(parts0
                + [r1[0, 0].reshape(NP, 64), r1[0, 1].reshape(NP, 64),
                   r1[1, 0].reshape(NP, 64), r1[1, 1].reshape(NP, 64)],
                w_cat, b_lin, relu=True)
    return out[:N]


# speedup vs baseline: 7.7726x; 1.1464x over previous
import functools

import jax
import jax.numpy as jnp
from jax import lax
from jax.experimental import pallas as pl
from jax.experimental.pallas import tpu as pltpu
from jax.experimental.pallas import tpu_sc as plsc

N = 10000
E = 320000
HEADS = 4
HID = 64
DF = 128
OUT = 128
NEG = 0.2

NC, NS, L = 2, 16, 16
NP = 10240
NP2 = 2 * NP
EPAD = 327680
EPT = EPAD // NS
LPT = NP // NS
EP3 = EPAD + NP

@functools.cache
def _mesh():
    return plsc.VectorSubcoreMesh(core_axis_name="c", subcore_axis_name="s",
                                  num_cores=NC, num_subcores=NS)


_BN = 512


def _proj_body(widths, relu, *refs):
    nparts = len(widths)
    parts = refs[:nparts]
    w_ref, b_ref, o_ref = refs[nparts], refs[nparts + 1], refs[nparts + 2]
    k0 = 0
    acc = None
    for i, w in enumerate(widths):
        term = jnp.dot(parts[i][...], w_ref[k0:k0 + w, :],
                       preferred_element_type=jnp.float32)
        acc = term if acc is None else acc + term
        k0 += w
    acc = acc + b_ref[...]
    if relu:
        acc = jnp.maximum(acc, 0.0)
    o_ref[...] = acc


def _proj(parts, w, b, relu=False):
    widths = tuple(p.shape[1] for p in parts)
    dout = w.shape[1]
    grid = (NP // _BN,)
    in_specs = [pl.BlockSpec((_BN, pw), lambda i: (i, 0)) for pw in widths]
    in_specs.append(pl.BlockSpec(w.shape, lambda i: (0, 0)))
    in_specs.append(pl.BlockSpec((1, dout), lambda i: (0, 0)))
    out_shape = jax.ShapeDtypeStruct((NP, dout), jnp.float32)
    out_spec = pl.BlockSpec((_BN, dout), lambda i: (i, 0))
    return pl.pallas_call(
        functools.partial(_proj_body, widths, relu),
        grid=grid,
        in_specs=in_specs,
        out_specs=out_spec,
        out_shape=out_shape,
    )(*parts, w, b.reshape(1, dout))


_CH0 = 2048


def _k0_body(eix, ea, la, sv, dv, eav, accs, accc, shared, ts, tt, sem):
    c = lax.axis_index("c")
    s = lax.axis_index("s")

    @pl.loop(0, NP // L)
    def _(i):
        z = jnp.zeros((L,), jnp.float32)
        accs[pl.ds(i * L, L)] = z
        accc[pl.ds(i * L, L)] = z

    base = s * EPT

    @pl.loop(0, EPT // _CH0)
    def _(k):
        off = base + k * _CH0
        pltpu.sync_copy(eix.at[0, pl.ds(off, _CH0)], sv)
        pltpu.sync_copy(eix.at[1, pl.ds(off, _CH0)], dv)
        pltpu.sync_copy(ea.at[pl.ds(off, _CH0)], eav)

        @pl.loop(0, _CH0 // L)
        def _(i):
            ds = pl.ds(i * L, L)
            svv = sv[ds]
            dvv = dv[ds]
            evv = eav[ds]
            m = svv != dvv
            val = jnp.where(m, evv, 0.0)
            cnt = jnp.where(m, 1.0, 0.0)
            plsc.addupdate_scatter(accs, [dvv], val)
            plsc.addupdate_scatter(accc, [dvv], cnt)

    pltpu.sync_copy(accs, shared.at[s, 0])
    pltpu.sync_copy(accc, shared.at[s, 1])
    plsc.subcore_barrier()

    r0 = s * LPT
    pltpu.sync_copy(shared.at[0, 0, pl.ds(r0, LPT)], accs.at[pl.ds(0, LPT)])
    pltpu.sync_copy(shared.at[0, 1, pl.ds(r0, LPT)], accc.at[pl.ds(0, LPT)])
    for t in range(1, NS):
        pltpu.sync_copy(shared.at[t, 0, pl.ds(r0, LPT)], ts)
        pltpu.sync_copy(shared.at[t, 1, pl.ds(r0, LPT)], tt)

        @pl.loop(0, LPT // L)
        def _(i):
            ds = pl.ds(i * L, L)
            accs[ds] = accs[ds] + ts[ds]
            accc[ds] = accc[ds] + tt[ds]

    @pl.loop(0, LPT // L)
    def _(i):
        ds = pl.ds(i * L, L)
        ts[ds] = accs[ds] / jnp.maximum(accc[ds], 1.0)

    pltpu.sync_copy(ts.at[pl.ds(0, LPT)], la.at[c, pl.ds(r0, LPT)])


def _k0(eix, ea):
    f = pl.kernel(
        _k0_body,
        out_type=jax.ShapeDtypeStruct((2, NP), jnp.float32),
        mesh=_mesh(),
        compiler_params=pltpu.CompilerParams(needs_layout_passes=False),
        scratch_types=[
            pltpu.VMEM((_CH0,), jnp.int32),
            pltpu.VMEM((_CH0,), jnp.int32),
            pltpu.VMEM((_CH0,), jnp.float32),
            pltpu.VMEM((NP,), jnp.float32),
            pltpu.VMEM((NP,), jnp.float32),
            pltpu.VMEM_SHARED((NS, 2, NP), jnp.float32),
            pltpu.VMEM((LPT,), jnp.float32),
            pltpu.VMEM((LPT,), jnp.float32),
            pltpu.SemaphoreType.DMA,
        ],
    )
    return f(eix, ea)


_CHA = 256


def _ka_edge_chunk(c, loop_phase, sv, dv, eav, gis, gid, A, B, a0st, a1st,
                   xlr, xrr, sem, sem2, we, at0, at1, cha):
    nsub = cha // 128
    descs = []
    for q in range(nsub):
        descs.append(pltpu.async_copy(
            xlr.at[gis.at[q]], A.at[pl.ds(q * 128, 128)], sem))
        descs.append(pltpu.async_copy(
            xrr.at[gid.at[q]], B.at[pl.ds(q * 128, 128)], sem2))
    for d in descs:
        d.wait()

    liota = lax.iota(jnp.int32, L)

    @pl.loop(0, cha // L)
    def _(i):
        ds = pl.ds(i * L, L)
        ev = eav[ds]
        svv = sv[ds]
        dvv = dv[ds]
        if loop_phase:
            badv = svv >= N
        else:
            badv = svv == dvv
        av0 = jnp.zeros((L,), jnp.float32)
        av1 = jnp.zeros((L,), jnp.float32)
        for lane in range(L):
            j = i * L + lane
            eaj = ev[lane]
            acc0 = jnp.zeros((L,), jnp.float32)
            acc1 = jnp.zeros((L,), jnp.float32)
            for kk in range(8):
                dsk = pl.ds(kk * L, L)
                m = A[j, dsk] + B[j, dsk] + eaj * we[kk]
                m = jnp.maximum(m, 0.0) + NEG * jnp.minimum(m, 0.0)
                if kk < 4:
                    acc0 = acc0 + m * at0[kk]
                else:
                    acc1 = acc1 + m * at1[kk - 4]
            al0 = jnp.sum(acc0)
            al1 = jnp.sum(acc1)
            av0 = av0 + jnp.where(liota == lane, al0, 0.0)
            av1 = av1 + jnp.where(liota == lane, al1, 0.0)
        a0st[ds] = jnp.where(badv, -1e30, av0)
        a1st[ds] = jnp.where(badv, -1e30, av1)


def _ka_body(xlr, xrr, eix, ea, la, we2, att2, alpha, mx,
             sv, dv, eav, gis, gid, A, B, a0st, a1st, wev, attv, mxv,
             shmx, sem, sem2):
    c = lax.axis_index("c")
    s = lax.axis_index("s")

    pltpu.sync_copy(we2.at[c], wev)
    pltpu.sync_copy(att2.at[c], attv)
    we = [wev[pl.ds(k * L, L)] for k in range(8)]
    at0 = [attv[0, pl.ds(k * L, L)] for k in range(4)]
    at1 = [attv[1, pl.ds(k * L, L)] for k in range(4)]

    neg = jnp.full((L,), -1e30, jnp.float32)

    base = s * EPT

    @pl.loop(0, EPT // _CHA, init_carry=(neg, neg))
    def real_mx(k, carry):
        off = base + k * _CHA
        pltpu.sync_copy(eix.at[0, pl.ds(off, _CHA)], sv)
        pltpu.sync_copy(eix.at[1, pl.ds(off, _CHA)], dv)
        pltpu.sync_copy(ea.at[pl.ds(off, _CHA)], eav)
        for q in range(_CHA // 128):
            @pl.loop(0, 128 // L)
            def _(i):
                ds = pl.ds(q * 128 + i * L, L)
                dsq = pl.ds(i * L, L)
                gis[q, dsq] = sv[ds] * 2 + c
                gid[q, dsq] = dv[ds] * 2 + c
        _ka_edge_chunk(c, False, sv, dv, eav, gis, gid, A, B, a0st, a1st,
                       xlr, xrr, sem, sem2, we, at0, at1, _CHA)
        pltpu.sync_copy(a0st, alpha.at[c, 0, pl.ds(off, _CHA)])
        pltpu.sync_copy(a1st, alpha.at[c, 1, pl.ds(off, _CHA)])
        m0, m1 = carry

        @pl.loop(0, _CHA // L, init_carry=(m0, m1))
        def mxf(i, cc):
            p0, p1 = cc
            ds = pl.ds(i * L, L)
            return (jnp.maximum(p0, a0st[ds]), jnp.maximum(p1, a1st[ds]))

        return mxf

    nb = s * LPT
    liota = lax.iota(jnp.int32, L)

    @pl.loop(0, LPT // 128, init_carry=real_mx)
    def loop_mx(k, carry):
        off_n = nb + k * 128
        pltpu.sync_copy(la.at[c, pl.ds(off_n, 128)], eav.at[pl.ds(0, 128)])

        @pl.loop(0, 128 // L)
        def _(i):
            nv = off_n + i * L + liota
            ncl = jnp.minimum(nv, N - 1)
            dsq = pl.ds(i * L, L)
            sv[dsq] = nv
            dv[dsq] = nv
            gis[0, dsq] = ncl * 2 + c
            gid[0, dsq] = ncl * 2 + c
        _ka_edge_chunk(c, True, sv, dv, eav, gis, gid, A, B, a0st, a1st,
                       xlr, xrr, sem, sem2, we, at0, at1, 128)
        aoff = EPAD + off_n
        pltpu.sync_copy(a0st.at[pl.ds(0, 128)], alpha.at[c, 0, pl.ds(aoff, 128)])
        pltpu.sync_copy(a1st.at[pl.ds(0, 128)], alpha.at[c, 1, pl.ds(aoff, 128)])
        m0, m1 = carry

        @pl.loop(0, 128 // L, init_carry=(m0, m1))
        def mxf(i, cc):
            p0, p1 = cc
            ds = pl.ds(i * L, L)
            return (jnp.maximum(p0, a0st[ds]), jnp.maximum(p1, a1st[ds]))

        return mxf

    m0, m1 = loop_mx
    mxv[0, pl.ds(0, L)] = m0
    mxv[1, pl.ds(0, L)] = m1
    pltpu.sync_copy(mxv, shmx.at[s])
    plsc.subcore_barrier()

    pltpu.sync_copy(shmx.at[0], mxv)
    r0 = mxv[0, pl.ds(0, L)]
    r1 = mxv[1, pl.ds(0, L)]
    for t in range(1, NS):
        pltpu.sync_copy(shmx.at[t], mxv)
        r0 = jnp.maximum(r0, mxv[0, pl.ds(0, L)])
        r1 = jnp.maximum(r1, mxv[1, pl.ds(0, L)])
    mxv[0, pl.ds(0, L)] = jnp.full((L,), jnp.max(r0), jnp.float32)
    mxv[1, pl.ds(0, L)] = jnp.full((L,), jnp.max(r1), jnp.float32)

    @pl.when(s == 0)
    def _():
        pltpu.sync_copy(mxv, mx.at[c])


def _ka(xlr, xrr, eix, ea, la, we2, att2):
    f = pl.kernel(
        _ka_body,
        out_type=[
            jax.ShapeDtypeStruct((2, 2, EP3), jnp.float32),
            jax.ShapeDtypeStruct((2, 2, L), jnp.float32),
        ],
        mesh=_mesh(),
        compiler_params=pltpu.CompilerParams(needs_layout_passes=False),
        scratch_types=[
            pltpu.VMEM((_CHA,), jnp.int32),
            pltpu.VMEM((_CHA,), jnp.int32),
            pltpu.VMEM((_CHA,), jnp.float32),
            pltpu.VMEM((_CHA // 128, 128), jnp.int32),
            pltpu.VMEM((_CHA // 128, 128), jnp.int32),
            pltpu.VMEM((_CHA, 128), jnp.float32),
            pltpu.VMEM((_CHA, 128), jnp.float32),
            pltpu.VMEM((_CHA,), jnp.float32),
            pltpu.VMEM((_CHA,), jnp.float32),
            pltpu.VMEM((128,), jnp.float32),
            pltpu.VMEM((2, 64), jnp.float32),
            pltpu.VMEM((2, L), jnp.float32),
            pltpu.VMEM_SHARED((NS, 2, L), jnp.float32),
            pltpu.SemaphoreType.DMA,
            pltpu.SemaphoreType.DMA,
        ],
    )
    return f(xlr, xrr, eix, ea, la, we2, att2)


_CHC = 128


def _kc_body(alpha, mx, eix, xlr, bias4, h,
             sv, dvi, gis, exv, A, As, denp, dred, zbuf, mxv, bv4, iov,
             acc, densh, sem):
    c = lax.axis_index("c")
    s = lax.axis_index("s")
    liota = lax.iota(jnp.int32, L)

    pltpu.sync_copy(mx.at[c], mxv)
    pltpu.sync_copy(bias4, bv4)

    @pl.loop(0, 80 // L)
    def _(i):
        iov[0, pl.ds(i * L, L)] = i * L + liota

    for p in range(2):
        mp = mxv[p, pl.ds(0, L)][0]
        hidx = c * 2 + p
        bq = [bv4[hidx, pl.ds(kk * L, L)] for kk in range(4)]

        @pl.loop(0, 80)
        def _(r):
            for kk in range(8):
                denp[r, pl.ds(kk * L, L)] = jnp.zeros((L,), jnp.float32)

        @pl.loop(0, 64)
        def _(j):
            for kk in range(8):
                As[j, pl.ds(kk * L, L)] = jnp.zeros((L,), jnp.float32)

        @pl.loop(0, 5)
        def _(r):
            for kk in range(8):
                zbuf[r, pl.ds(kk * L, L)] = jnp.zeros((L,), jnp.float32)
        for q in range(5):
            pltpu.sync_copy(As.at[pl.ds(0, 64)],
                            acc.at[pl.ds(s * 320 + q * 64, 64)])
        pltpu.sync_copy(zbuf, densh.at[pl.ds(s * 5, 5)])
        plsc.subcore_barrier()

        def do_chunk(off, aoff, loop_phase):
            cha = _CHC
            if loop_phase:
                @pl.loop(0, cha // L)
                def _(i):
                    nv = off + i * L + liota
                    ncl = jnp.minimum(nv, N - 1)
                    sv[pl.ds(i * L, L)] = ncl
            else:
                pltpu.sync_copy(eix.at[0, pl.ds(off, cha)], sv)
            pltpu.sync_copy(alpha.at[c, p, pl.ds(aoff, cha)], exv)

            @pl.loop(0, cha // L)
            def _(i):
                ds = pl.ds(i * L, L)
                gis[ds] = sv[ds] * 2 + c
                exv[ds] = jnp.exp(exv[ds] - mp)

            if not loop_phase:
                pltpu.sync_copy(eix.at[1, pl.ds(off, cha)], sv)

            gat = pltpu.async_copy(xlr.at[gis], A, sem)

            @pl.loop(0, cha // L)
            def _(i):
                ds = pl.ds(i * L, L)
                dvv = sv[ds]
                plsc.addupdate_scatter(denp, [dvv // 128, dvv % 128], exv[ds])
                dvi[0, ds] = dvv // 2

            gat.wait()

            @pl.loop(0, cha // L)
            def _(i):
                ds = pl.ds(i * L, L)
                ev = exv[ds]
                pf = (sv[ds] % 2).astype(jnp.float32)
                for lane in range(L):
                    j = i * L + lane
                    e = ev[lane]
                    p1 = pf[lane]
                    e0 = e * (1.0 - p1)
                    e1 = e * p1
                    for kk in range(4):
                        v = A[j, pl.ds(p * 64 + kk * L, L)]
                        As[j, pl.ds(kk * L, L)] = v * e0
                        As[j, pl.ds(64 + kk * L, L)] = v * e1

            pltpu.sync_copy(As, acc.at[dvi.at[0]], add=True)

        base = s * EPT

        @pl.loop(0, EPT // _CHC)
        def _(k):
            off = base + k * _CHC
            do_chunk(off, off, False)

        nb = s * LPT

        @pl.loop(0, LPT // _CHC)
        def _(k):
            off = nb + k * _CHC
            do_chunk(off, EPAD + off, True)

        pltpu.sync_copy(denp, densh.at[iov.at[0]], add=True)
        plsc.subcore_barrier()

        pltpu.sync_copy(densh.at[pl.ds(s * 5, 5)], dred)
        for q in range(5):
            rr = s * 320 + q * 64
            pltpu.sync_copy(acc.at[pl.ds(rr, 64)], As.at[pl.ds(0, 64)])

            @pl.loop(0, 8)
            def _(g2):
                invv = 1.0 / (dred[q, pl.ds(g2 * L, L)] + 1e-16)
                for jj in range(8):
                    j = g2 * 8 + jj
                    i0 = invv[2 * jj]
                    i1 = invv[2 * jj + 1]
                    for kk in range(4):
                        d0 = pl.ds(kk * L, L)
                        d1 = pl.ds(64 + kk * L, L)
                        As[j, d0] = jnp.maximum(As[j, d0] * i0 + bq[kk], 0.0)
                        As[j, d1] = jnp.maximum(As[j, d1] * i1 + bq[kk], 0.0)

            pltpu.sync_copy(As.at[pl.ds(0, 64)], h.at[c, p, pl.ds(rr, 64)])
        plsc.subcore_barrier()


def _kc(alpha, mx, eix, xlr, bias4):
    f = pl.kernel(
        _kc_body,
        out_type=jax.ShapeDtypeStruct((2, 2, NP // 2, 128), jnp.float32),
        mesh=_mesh(),
        compiler_params=pltpu.CompilerParams(needs_layout_passes=False),
        scratch_types=[
            pltpu.VMEM((_CHC,), jnp.int32),
            pltpu.VMEM((1, 128), jnp.int32),
            pltpu.VMEM((_CHC,), jnp.int32),
            pltpu.VMEM((_CHC,), jnp.float32),
            pltpu.VMEM((_CHC, 128), jnp.float32),
            pltpu.VMEM((_CHC, 128), jnp.float32),
            pltpu.VMEM((80, 128), jnp.float32),
            pltpu.VMEM((5, 128), jnp.float32),
            pltpu.VMEM((5, 128), jnp.float32),
            pltpu.VMEM((2, L), jnp.float32),
            pltpu.VMEM((4, 64), jnp.float32),
            pltpu.VMEM((1, 80), jnp.int32),
            pltpu.VMEM_SHARED((NP // 2, 128), jnp.float32),
            pltpu.VMEM_SHARED((80, 128), jnp.float32),
            pltpu.SemaphoreType.DMA,
        ],
    )
    return f(alpha, mx, eix, xlr, bias4)


def kernel(x, edge_index, edge_attr,
           W_l0, b_l0, W_r0, b_r0, W_e0, att0, bias0,
           W_l1, b_l1, W_r1, b_r1, W_e1, att1, bias1,
           W_lin, b_lin):
    x = x.astype(jnp.float32)
    ea = edge_attr.astype(jnp.float32).reshape(E)
    ea = jnp.pad(ea, (0, EPAD - E))
    eix = jnp.pad(edge_index, ((0, 0), (0, EPAD - E)))

    xp = jnp.pad(x, ((0, NP - N), (0, 0)))

    la = _k0(eix, ea)

    we2_0 = W_e0.reshape(2, 128)
    att2_0 = att0.reshape(2, 2, HID)
    bias4_0 = bias0.reshape(4, 64)
    we2_1 = W_e1.reshape(2, 128)
    att2_1 = att1.reshape(2, 2, HID)
    bias4_1 = bias1.reshape(4, 64)

    xl0 = _proj([xp], W_l0, b_l0)
    xr0 = _proj([xp], W_r0, b_r0)
    alpha0, mx0 = _ka(xl0.reshape(NP2, 128), xr0.reshape(NP2, 128),
                      eix, ea, la, we2_0, att2_0)
    h0 = _kc(alpha0, mx0, eix, xl0.reshape(NP2, 128), bias4_0)

    parts0 = [h0[0, 0].reshape(NP, 64), h0[0, 1].reshape(NP, 64),
              h0[1, 0].reshape(NP, 64), h0[1, 1].reshape(NP, 64)]
    xl1 = _proj(parts0, W_l1, b_l1)
    xr1 = _proj(parts0, W_r1, b_r1)
    alpha1, mx1 = _ka(xl1.reshape(NP2, 128), xr1.reshape(NP2, 128),
                      eix, ea, la, we2_1, att2_1)
    r1 = _kc(alpha1, mx1, eix, xl1.reshape(NP2, 128), bias4_1)

    w_cat = jnp.concatenate([W_lin, W_lin], axis=0)
    out = _proj(parts0
                + [r1[0, 0].reshape(NP, 64), r1[0, 1].reshape(NP, 64),
                   r1[1, 0].reshape(NP, 64), r1[1, 1].reshape(NP, 64)],
                w_cat, b_lin, relu=True)
    return out[:N]


# speedup vs baseline: 9.0872x; 1.1691x over previous
import functools

import jax
import jax.numpy as jnp
from jax import lax
from jax.experimental import pallas as pl
from jax.experimental.pallas import tpu as pltpu
from jax.experimental.pallas import tpu_sc as plsc

N = 10000
E = 320000
HEADS = 4
HID = 64
DF = 128
OUT = 128
NEG = 0.2

NC, NS, L = 2, 16, 16
NP = 10240
NP2 = 2 * NP
EPAD = 327680
EPT = EPAD // NS
LPT = NP // NS
EP3 = EPAD + NP

@functools.cache
def _mesh():
    return plsc.VectorSubcoreMesh(core_axis_name="c", subcore_axis_name="s",
                                  num_cores=NC, num_subcores=NS)


_BN = 512


def _proj_body(widths, relu, *refs):
    nparts = len(widths)
    parts = refs[:nparts]
    w_ref, b_ref, o_ref = refs[nparts], refs[nparts + 1], refs[nparts + 2]
    k0 = 0
    acc = None
    for i, w in enumerate(widths):
        term = jnp.dot(parts[i][...], w_ref[k0:k0 + w, :],
                       preferred_element_type=jnp.float32)
        acc = term if acc is None else acc + term
        k0 += w
    acc = acc + b_ref[...]
    if relu:
        acc = jnp.maximum(acc, 0.0)
    o_ref[...] = acc


def _proj(parts, w, b, relu=False):
    widths = tuple(p.shape[1] for p in parts)
    dout = w.shape[1]
    grid = (NP // _BN,)
    in_specs = [pl.BlockSpec((_BN, pw), lambda i: (i, 0)) for pw in widths]
    in_specs.append(pl.BlockSpec(w.shape, lambda i: (0, 0)))
    in_specs.append(pl.BlockSpec((1, dout), lambda i: (0, 0)))
    out_shape = jax.ShapeDtypeStruct((NP, dout), jnp.float32)
    out_spec = pl.BlockSpec((_BN, dout), lambda i: (i, 0))
    return pl.pallas_call(
        functools.partial(_proj_body, widths, relu),
        grid=grid,
        in_specs=in_specs,
        out_specs=out_spec,
        out_shape=out_shape,
    )(*parts, w, b.reshape(1, dout))


_CH0 = 2048


def _k0_body(eix, ea, la, sv, dv, eav, accs, accc, shared, ts, tt, sem):
    c = lax.axis_index("c")
    s = lax.axis_index("s")

    @pl.loop(0, NP // L)
    def _(i):
        z = jnp.zeros((L,), jnp.float32)
        accs[pl.ds(i * L, L)] = z
        accc[pl.ds(i * L, L)] = z

    base = s * EPT

    @pl.loop(0, EPT // _CH0)
    def _(k):
        off = base + k * _CH0
        pltpu.sync_copy(eix.at[0, pl.ds(off, _CH0)], sv)
        pltpu.sync_copy(eix.at[1, pl.ds(off, _CH0)], dv)
        pltpu.sync_copy(ea.at[pl.ds(off, _CH0)], eav)

        @pl.loop(0, _CH0 // L)
        def _(i):
            ds = pl.ds(i * L, L)
            svv = sv[ds]
            dvv = dv[ds]
            evv = eav[ds]
            m = svv != dvv
            val = jnp.where(m, evv, 0.0)
            cnt = jnp.where(m, 1.0, 0.0)
            plsc.addupdate_scatter(accs, [dvv], val)
            plsc.addupdate_scatter(accc, [dvv], cnt)

    pltpu.sync_copy(accs, shared.at[s, 0])
    pltpu.sync_copy(accc, shared.at[s, 1])
    plsc.subcore_barrier()

    r0 = s * LPT
    pltpu.sync_copy(shared.at[0, 0, pl.ds(r0, LPT)], accs.at[pl.ds(0, LPT)])
    pltpu.sync_copy(shared.at[0, 1, pl.ds(r0, LPT)], accc.at[pl.ds(0, LPT)])
    for t in range(1, NS):
        pltpu.sync_copy(shared.at[t, 0, pl.ds(r0, LPT)], ts)
        pltpu.sync_copy(shared.at[t, 1, pl.ds(r0, LPT)], tt)

        @pl.loop(0, LPT // L)
        def _(i):
            ds = pl.ds(i * L, L)
            accs[ds] = accs[ds] + ts[ds]
            accc[ds] = accc[ds] + tt[ds]

    @pl.loop(0, LPT // L)
    def _(i):
        ds = pl.ds(i * L, L)
        ts[ds] = accs[ds] / jnp.maximum(accc[ds], 1.0)

    pltpu.sync_copy(ts.at[pl.ds(0, LPT)], la.at[c, pl.ds(r0, LPT)])


def _k0(eix, ea):
    f = pl.kernel(
        _k0_body,
        out_type=jax.ShapeDtypeStruct((2, NP), jnp.float32),
        mesh=_mesh(),
        compiler_params=pltpu.CompilerParams(needs_layout_passes=False),
        scratch_types=[
            pltpu.VMEM((_CH0,), jnp.int32),
            pltpu.VMEM((_CH0,), jnp.int32),
            pltpu.VMEM((_CH0,), jnp.float32),
            pltpu.VMEM((NP,), jnp.float32),
            pltpu.VMEM((NP,), jnp.float32),
            pltpu.VMEM_SHARED((NS, 2, NP), jnp.float32),
            pltpu.VMEM((LPT,), jnp.float32),
            pltpu.VMEM((LPT,), jnp.float32),
            pltpu.SemaphoreType.DMA,
        ],
    )
    return f(eix, ea)


_CHA = 256


def _ka_edge_chunk(c, loop_phase, sv, dv, eav, gis, gid, A, B, a0st, a1st,
                   xlr, xrr, sem, sem2, we, at0, at1, cha):
    nsub = cha // 128
    descs = []
    for q in range(nsub):
        descs.append(pltpu.async_copy(
            xlr.at[gis.at[q]], A.at[pl.ds(q * 128, 128)], sem))
        descs.append(pltpu.async_copy(
            xrr.at[gid.at[q]], B.at[pl.ds(q * 128, 128)], sem2))
    for d in descs:
        d.wait()

    liota = lax.iota(jnp.int32, L)

    @plsc.parallel_loop(0, cha // L, unroll=2)
    def _(i):
        ds = pl.ds(i * L, L)
        ev = eav[ds]
        svv = sv[ds]
        dvv = dv[ds]
        if loop_phase:
            badv = svv >= N
        else:
            badv = svv == dvv
        av0 = jnp.zeros((L,), jnp.float32)
        av1 = jnp.zeros((L,), jnp.float32)
        for lane in range(L):
            j = i * L + lane
            eaj = ev[lane]
            acc0 = jnp.zeros((L,), jnp.float32)
            acc1 = jnp.zeros((L,), jnp.float32)
            for kk in range(8):
                dsk = pl.ds(kk * L, L)
                m = A[j, dsk] + B[j, dsk] + eaj * we[kk]
                m = jnp.maximum(m, 0.0) + NEG * jnp.minimum(m, 0.0)
                if kk < 4:
                    acc0 = acc0 + m * at0[kk]
                else:
                    acc1 = acc1 + m * at1[kk - 4]
            al0 = jnp.sum(acc0)
            al1 = jnp.sum(acc1)
            av0 = av0 + jnp.where(liota == lane, al0, 0.0)
            av1 = av1 + jnp.where(liota == lane, al1, 0.0)
        a0st[ds] = jnp.where(badv, -1e30, av0)
        a1st[ds] = jnp.where(badv, -1e30, av1)


def _ka_body(xlr, xrr, eix, ea, la, we2, att2, alpha, mx,
             sv, dv, eav, gis, gid, A, B, a0st, a1st, wev, attv, mxv,
             shmx, sem, sem2):
    c = lax.axis_index("c")
    s = lax.axis_index("s")

    pltpu.sync_copy(we2.at[c], wev)
    pltpu.sync_copy(att2.at[c], attv)
    we = [wev[pl.ds(k * L, L)] for k in range(8)]
    at0 = [attv[0, pl.ds(k * L, L)] for k in range(4)]
    at1 = [attv[1, pl.ds(k * L, L)] for k in range(4)]

    neg = jnp.full((L,), -1e30, jnp.float32)

    base = s * EPT

    @pl.loop(0, EPT // _CHA, init_carry=(neg, neg))
    def real_mx(k, carry):
        off = base + k * _CHA
        pltpu.sync_copy(eix.at[0, pl.ds(off, _CHA)], sv)
        pltpu.sync_copy(eix.at[1, pl.ds(off, _CHA)], dv)
        pltpu.sync_copy(ea.at[pl.ds(off, _CHA)], eav)
        for q in range(_CHA // 128):
            @pl.loop(0, 128 // L)
            def _(i):
                ds = pl.ds(q * 128 + i * L, L)
                dsq = pl.ds(i * L, L)
                gis[q, dsq] = sv[ds] * 2 + c
                gid[q, dsq] = dv[ds] * 2 + c
        _ka_edge_chunk(c, False, sv, dv, eav, gis, gid, A, B, a0st, a1st,
                       xlr, xrr, sem, sem2, we, at0, at1, _CHA)
        pltpu.sync_copy(a0st, alpha.at[c, 0, pl.ds(off, _CHA)])
        pltpu.sync_copy(a1st, alpha.at[c, 1, pl.ds(off, _CHA)])
        m0, m1 = carry

        @pl.loop(0, _CHA // L, init_carry=(m0, m1))
        def mxf(i, cc):
            p0, p1 = cc
            ds = pl.ds(i * L, L)
            return (jnp.maximum(p0, a0st[ds]), jnp.maximum(p1, a1st[ds]))

        return mxf

    nb = s * LPT
    liota = lax.iota(jnp.int32, L)

    @pl.loop(0, LPT // 128, init_carry=real_mx)
    def loop_mx(k, carry):
        off_n = nb + k * 128
        pltpu.sync_copy(la.at[c, pl.ds(off_n, 128)], eav.at[pl.ds(0, 128)])

        @pl.loop(0, 128 // L)
        def _(i):
            nv = off_n + i * L + liota
            ncl = jnp.minimum(nv, N - 1)
            dsq = pl.ds(i * L, L)
            sv[dsq] = nv
            dv[dsq] = nv
            gis[0, dsq] = ncl * 2 + c
            gid[0, dsq] = ncl * 2 + c
        _ka_edge_chunk(c, True, sv, dv, eav, gis, gid, A, B, a0st, a1st,
                       xlr, xrr, sem, sem2, we, at0, at1, 128)
        aoff = EPAD + off_n
        pltpu.sync_copy(a0st.at[pl.ds(0, 128)], alpha.at[c, 0, pl.ds(aoff, 128)])
        pltpu.sync_copy(a1st.at[pl.ds(0, 128)], alpha.at[c, 1, pl.ds(aoff, 128)])
        m0, m1 = carry

        @pl.loop(0, 128 // L, init_carry=(m0, m1))
        def mxf(i, cc):
            p0, p1 = cc
            ds = pl.ds(i * L, L)
            return (jnp.maximum(p0, a0st[ds]), jnp.maximum(p1, a1st[ds]))

        return mxf

    m0, m1 = loop_mx
    mxv[0, pl.ds(0, L)] = m0
    mxv[1, pl.ds(0, L)] = m1
    pltpu.sync_copy(mxv, shmx.at[s])
    plsc.subcore_barrier()

    pltpu.sync_copy(shmx.at[0], mxv)
    r0 = mxv[0, pl.ds(0, L)]
    r1 = mxv[1, pl.ds(0, L)]
    for t in range(1, NS):
        pltpu.sync_copy(shmx.at[t], mxv)
        r0 = jnp.maximum(r0, mxv[0, pl.ds(0, L)])
        r1 = jnp.maximum(r1, mxv[1, pl.ds(0, L)])
    mxv[0, pl.ds(0, L)] = jnp.full((L,), jnp.max(r0), jnp.float32)
    mxv[1, pl.ds(0, L)] = jnp.full((L,), jnp.max(r1), jnp.float32)

    @pl.when(s == 0)
    def _():
        pltpu.sync_copy(mxv, mx.at[c])


def _ka(xlr, xrr, eix, ea, la, we2, att2):
    f = pl.kernel(
        _ka_body,
        out_type=[
            jax.ShapeDtypeStruct((2, 2, EP3), jnp.float32),
            jax.ShapeDtypeStruct((2, 2, L), jnp.float32),
        ],
        mesh=_mesh(),
        compiler_params=pltpu.CompilerParams(needs_layout_passes=False),
        scratch_types=[
            pltpu.VMEM((_CHA,), jnp.int32),
            pltpu.VMEM((_CHA,), jnp.int32),
            pltpu.VMEM((_CHA,), jnp.float32),
            pltpu.VMEM((_CHA // 128, 128), jnp.int32),
            pltpu.VMEM((_CHA // 128, 128), jnp.int32),
            pltpu.VMEM((_CHA, 128), jnp.float32),
            pltpu.VMEM((_CHA, 128), jnp.float32),
            pltpu.VMEM((_CHA,), jnp.float32),
            pltpu.VMEM((_CHA,), jnp.float32),
            pltpu.VMEM((128,), jnp.float32),
            pltpu.VMEM((2, 64), jnp.float32),
            pltpu.VMEM((2, L), jnp.float32),
            pltpu.VMEM_SHARED((NS, 2, L), jnp.float32),
            pltpu.SemaphoreType.DMA,
            pltpu.SemaphoreType.DMA,
        ],
    )
    return f(xlr, xrr, eix, ea, la, we2, att2)


_CHC = 128


def _kc_body(alpha, mx, eix, xlr, bias4, h,
             sv, dvi, gis, exv, A, As, denp, dred, zbuf, mxv, bv4, iov,
             acc, densh, sem):
    c = lax.axis_index("c")
    s = lax.axis_index("s")
    liota = lax.iota(jnp.int32, L)

    pltpu.sync_copy(mx.at[c], mxv)
    pltpu.sync_copy(bias4, bv4)

    @pl.loop(0, 80 // L)
    def _(i):
        iov[0, pl.ds(i * L, L)] = i * L + liota

    for p in range(2):
        mp = mxv[p, pl.ds(0, L)][0]
        hidx = c * 2 + p
        bq = [bv4[hidx, pl.ds(kk * L, L)] for kk in range(4)]

        @pl.loop(0, 80)
        def _(r):
            for kk in range(8):
                denp[r, pl.ds(kk * L, L)] = jnp.zeros((L,), jnp.float32)

        @pl.loop(0, 64)
        def _(j):
            for kk in range(8):
                As[j, pl.ds(kk * L, L)] = jnp.zeros((L,), jnp.float32)

        @pl.loop(0, 5)
        def _(r):
            for kk in range(8):
                zbuf[r, pl.ds(kk * L, L)] = jnp.zeros((L,), jnp.float32)
        for q in range(5):
            pltpu.sync_copy(As.at[pl.ds(0, 64)],
                            acc.at[pl.ds(s * 320 + q * 64, 64)])
        pltpu.sync_copy(zbuf, densh.at[pl.ds(s * 5, 5)])
        plsc.subcore_barrier()

        def do_chunk(off, aoff, loop_phase):
            cha = _CHC
            if loop_phase:
                @pl.loop(0, cha // L)
                def _(i):
                    nv = off + i * L + liota
                    ncl = jnp.minimum(nv, N - 1)
                    sv[pl.ds(i * L, L)] = ncl
            else:
                pltpu.sync_copy(eix.at[0, pl.ds(off, cha)], sv)
            pltpu.sync_copy(alpha.at[c, p, pl.ds(aoff, cha)], exv)

            @pl.loop(0, cha // L)
            def _(i):
                ds = pl.ds(i * L, L)
                gis[ds] = sv[ds] * 2 + c
                exv[ds] = jnp.exp(exv[ds] - mp)

            if not loop_phase:
                pltpu.sync_copy(eix.at[1, pl.ds(off, cha)], sv)

            gat = pltpu.async_copy(xlr.at[gis], A, sem)

            @pl.loop(0, cha // L)
            def _(i):
                ds = pl.ds(i * L, L)
                dvv = sv[ds]
                plsc.addupdate_scatter(denp, [dvv // 128, dvv % 128], exv[ds])
                dvi[0, ds] = dvv // 2

            gat.wait()

            @plsc.parallel_loop(0, cha // L, unroll=2)
            def _(i):
                ds = pl.ds(i * L, L)
                ev = exv[ds]
                pf = (sv[ds] % 2).astype(jnp.float32)
                for lane in range(L):
                    j = i * L + lane
                    e = ev[lane]
                    p1 = pf[lane]
                    e0 = e * (1.0 - p1)
                    e1 = e * p1
                    for kk in range(4):
                        v = A[j, pl.ds(p * 64 + kk * L, L)]
                        As[j, pl.ds(kk * L, L)] = v * e0
                        As[j, pl.ds(64 + kk * L, L)] = v * e1

            pltpu.sync_copy(As, acc.at[dvi.at[0]], add=True)

        base = s * EPT

        @pl.loop(0, EPT // _CHC)
        def _(k):
            off = base + k * _CHC
            do_chunk(off, off, False)

        nb = s * LPT

        @pl.loop(0, LPT // _CHC)
        def _(k):
            off = nb + k * _CHC
            do_chunk(off, EPAD + off, True)

        pltpu.sync_copy(denp, densh.at[iov.at[0]], add=True)
        plsc.subcore_barrier()

        pltpu.sync_copy(densh.at[pl.ds(s * 5, 5)], dred)
        for q in range(5):
            rr = s * 320 + q * 64
            pltpu.sync_copy(acc.at[pl.ds(rr, 64)], As.at[pl.ds(0, 64)])

            @pl.loop(0, 8)
            def _(g2):
                invv = 1.0 / (dred[q, pl.ds(g2 * L, L)] + 1e-16)
                for jj in range(8):
                    j = g2 * 8 + jj
                    i0 = invv[2 * jj]
                    i1 = invv[2 * jj + 1]
                    for kk in range(4):
                        d0 = pl.ds(kk * L, L)
                        d1 = pl.ds(64 + kk * L, L)
                        As[j, d0] = jnp.maximum(As[j, d0] * i0 + bq[kk], 0.0)
                        As[j, d1] = jnp.maximum(As[j, d1] * i1 + bq[kk], 0.0)

            pltpu.sync_copy(As.at[pl.ds(0, 64)], h.at[c, p, pl.ds(rr, 64)])
        plsc.subcore_barrier()


def _kc(alpha, mx, eix, xlr, bias4):
    f = pl.kernel(
        _kc_body,
        out_type=jax.ShapeDtypeStruct((2, 2, NP // 2, 128), jnp.float32),
        mesh=_mesh(),
        compiler_params=pltpu.CompilerParams(needs_layout_passes=False),
        scratch_types=[
            pltpu.VMEM((_CHC,), jnp.int32),
            pltpu.VMEM((1, 128), jnp.int32),
            pltpu.VMEM((_CHC,), jnp.int32),
            pltpu.VMEM((_CHC,), jnp.float32),
            pltpu.VMEM((_CHC, 128), jnp.float32),
            pltpu.VMEM((_CHC, 128), jnp.float32),
            pltpu.VMEM((80, 128), jnp.float32),
            pltpu.VMEM((5, 128), jnp.float32),
            pltpu.VMEM((5, 128), jnp.float32),
            pltpu.VMEM((2, L), jnp.float32),
            pltpu.VMEM((4, 64), jnp.float32),
            pltpu.VMEM((1, 80), jnp.int32),
            pltpu.VMEM_SHARED((NP // 2, 128), jnp.float32),
            pltpu.VMEM_SHARED((80, 128), jnp.float32),
            pltpu.SemaphoreType.DMA,
        ],
    )
    return f(alpha, mx, eix, xlr, bias4)


def kernel(x, edge_index, edge_attr,
           W_l0, b_l0, W_r0, b_r0, W_e0, att0, bias0,
           W_l1, b_l1, W_r1, b_r1, W_e1, att1, bias1,
           W_lin, b_lin):
    x = x.astype(jnp.float32)
    ea = edge_attr.astype(jnp.float32).reshape(E)
    ea = jnp.pad(ea, (0, EPAD - E))
    eix = jnp.pad(edge_index, ((0, 0), (0, EPAD - E)))

    xp = jnp.pad(x, ((0, NP - N), (0, 0)))

    la = _k0(eix, ea)

    we2_0 = W_e0.reshape(2, 128)
    att2_0 = att0.reshape(2, 2, HID)
    bias4_0 = bias0.reshape(4, 64)
    we2_1 = W_e1.reshape(2, 128)
    att2_1 = att1.reshape(2, 2, HID)
    bias4_1 = bias1.reshape(4, 64)

    xl0 = _proj([xp], W_l0, b_l0)
    xr0 = _proj([xp], W_r0, b_r0)
    alpha0, mx0 = _ka(xl0.reshape(NP2, 128), xr0.reshape(NP2, 128),
                      eix, ea, la, we2_0, att2_0)
    h0 = _kc(alpha0, mx0, eix, xl0.reshape(NP2, 128), bias4_0)

    parts0 = [h0[0, 0].reshape(NP, 64), h0[0, 1].reshape(NP, 64),
              h0[1, 0].reshape(NP, 64), h0[1, 1].reshape(NP, 64)]
    xl1 = _proj(parts0, W_l1, b_l1)
    xr1 = _proj(parts0, W_r1, b_r1)
    alpha1, mx1 = _ka(xl1.reshape(NP2, 128), xr1.reshape(NP2, 128),
                      eix, ea, la, we2_1, att2_1)
    r1 = _kc(alpha1, mx1, eix, xl1.reshape(NP2, 128), bias4_1)

    w_cat = jnp.concatenate([W_lin, W_lin], axis=0)
    out = _proj(parts0
                + [r1[0, 0].reshape(NP, 64), r1[0, 1].reshape(NP, 64),
                   r1[1, 0].reshape(NP, 64), r1[1, 1].reshape(NP, 64)],
                w_cat, b_lin, relu=True)
    return out[:N]
